# Initial kernel scaffold; baseline (speedup 1.0000x reference)
import jax
import jax.numpy as jnp
from jax.experimental import pallas as pl


def kernel(z, edge_index):
    raise NotImplementedError("write your pallas kernel here")


# speedup vs baseline: 4.6839x; 4.6839x over previous
import functools

import jax
import jax.numpy as jnp
from jax import lax
from jax.experimental import pallas as pl
from jax.experimental.pallas import tpu as pltpu
from jax.experimental.pallas import tpu_sc as plsc

N_NODES = 10000
D_FEAT = 128
N_EDGES = 320000

NC = 2
NS = 16
NW = NC * NS
EDGES_PER_WORKER = N_EDGES // NW
CHUNK = 80
NCHUNK = EDGES_PER_WORKER // CHUNK
GROUPS = CHUNK // 16


def _tanh_neg_sqrt(acc):
    x = jnp.maximum(acc, jnp.float32(1e-30))
    i = plsc.bitcast(x, jnp.int32)
    i = jnp.int32(0x5F3759DF) - (i >> 1)
    y = plsc.bitcast(i, jnp.float32)
    half_x = jnp.float32(0.5) * x
    for _ in range(3):
        y = y * (jnp.float32(1.5) - half_x * y * y)
    dist = x * y
    u = jnp.exp(jnp.float32(-2.0) * dist)
    return (u - jnp.float32(1.0)) / (u + jnp.float32(1.0))


@functools.partial(
    pl.kernel,
    mesh=plsc.VectorSubcoreMesh(core_axis_name="c", subcore_axis_name="s"),
    out_type=jax.ShapeDtypeStruct((NW, NCHUNK, CHUNK), jnp.float32),
    compiler_params=pltpu.CompilerParams(needs_layout_passes=False),
    scratch_types=[
        pltpu.VMEM((NCHUNK, CHUNK), jnp.int32),
        pltpu.VMEM((NCHUNK, CHUNK), jnp.int32),
        pltpu.VMEM((CHUNK, D_FEAT), jnp.float32),
        pltpu.VMEM((CHUNK, D_FEAT), jnp.float32),
        pltpu.VMEM((CHUNK,), jnp.float32),
        pltpu.SemaphoreType.DMA,
    ],
)
def _sc_kernel(src_hbm, dst_hbm, z_hbm, out_hbm,
               idx_s, idx_d, rows_s, rows_d, outb, sem):
    wid = lax.axis_index("s") * NC + lax.axis_index("c")
    pltpu.sync_copy(src_hbm.at[wid], idx_s)
    pltpu.sync_copy(dst_hbm.at[wid], idx_d)

    lane = lax.iota(jnp.int32, 16)

    def chunk_body(j, carry):
        a = pltpu.async_copy(z_hbm.at[idx_s.at[j]], rows_s, sem)
        b = pltpu.async_copy(z_hbm.at[idx_d.at[j]], rows_d, sem)
        a.wait()
        b.wait()
        for g in range(GROUPS):

            def edge_body(e16, vec):
                e = jnp.int32(g * 16) + e16
                acc = jnp.zeros((16,), jnp.float32)
                for k in range(D_FEAT // 16):
                    s = rows_s[e, pl.ds(k * 16, 16)]
                    d = rows_d[e, pl.ds(k * 16, 16)]
                    t = s - d + jnp.float32(1e-6)
                    acc = acc + t * t
                tot = jnp.sum(acc)
                return jnp.where(lane == e16, tot, vec)

            vec = lax.fori_loop(0, 16, edge_body,
                                jnp.zeros((16,), jnp.float32), unroll=4)
            outb[pl.ds(g * 16, 16)] = _tanh_neg_sqrt(vec)
        pltpu.sync_copy(outb, out_hbm.at[wid, j])
        return carry

    lax.fori_loop(0, NCHUNK, chunk_body, jnp.int32(0))


def kernel(z, edge_index):
    ei = edge_index.astype(jnp.int32)
    src = ei[0].reshape(NW, NCHUNK, CHUNK)
    dst = ei[1].reshape(NW, NCHUNK, CHUNK)
    out = _sc_kernel(src, dst, z)
    return out.reshape(N_EDGES)


# speedup vs baseline: 6.8315x; 1.4585x over previous
import functools

import jax
import jax.numpy as jnp
from jax import lax
from jax.experimental import pallas as pl
from jax.experimental.pallas import tpu as pltpu
from jax.experimental.pallas import tpu_sc as plsc

N_NODES = 10000
D_FEAT = 128
N_EDGES = 320000

NC = 2
NS = 16
NW = NC * NS
EDGES_PER_WORKER = N_EDGES // NW
CHUNK = 80
NCHUNK = EDGES_PER_WORKER // CHUNK
GROUPS = CHUNK // 16


def _tanh_neg_sqrt(acc):
    x = jnp.maximum(acc, jnp.float32(1e-30))
    i = plsc.bitcast(x, jnp.int32)
    i = jnp.int32(0x5F3759DF) - (i >> 1)
    y = plsc.bitcast(i, jnp.float32)
    half_x = jnp.float32(0.5) * x
    for _ in range(3):
        y = y * (jnp.float32(1.5) - half_x * y * y)
    dist = x * y
    u = jnp.exp(jnp.float32(-2.0) * dist)
    return (u - jnp.float32(1.0)) / (u + jnp.float32(1.0))


@functools.partial(
    pl.kernel,
    mesh=plsc.VectorSubcoreMesh(core_axis_name="c", subcore_axis_name="s"),
    out_type=jax.ShapeDtypeStruct((N_EDGES,), jnp.float32),
    compiler_params=pltpu.CompilerParams(needs_layout_passes=False),
    scratch_types=[
        pltpu.VMEM((NCHUNK, CHUNK), jnp.int32),
        pltpu.VMEM((NCHUNK, CHUNK), jnp.int32),
        pltpu.VMEM((CHUNK, D_FEAT), jnp.float32),
        pltpu.VMEM((CHUNK, D_FEAT), jnp.float32),
        pltpu.VMEM((CHUNK, D_FEAT), jnp.float32),
        pltpu.VMEM((CHUNK, D_FEAT), jnp.float32),
        pltpu.VMEM((CHUNK,), jnp.float32),
        pltpu.SemaphoreType.DMA,
        pltpu.SemaphoreType.DMA,
    ],
)
def _sc_kernel(src_hbm, dst_hbm, z_hbm, out_hbm,
               idx_s, idx_d, rows_sa, rows_da, rows_sb, rows_db,
               outb, sem_a, sem_b):
    wid = lax.axis_index("s") * NC + lax.axis_index("c")
    pltpu.sync_copy(src_hbm.at[wid], idx_s)
    pltpu.sync_copy(dst_hbm.at[wid], idx_d)

    lane = lax.iota(jnp.int32, 16)

    def issue(j, rows_sx, rows_dx, semx):
        pltpu.async_copy(z_hbm.at[idx_s.at[j]], rows_sx, semx)
        pltpu.async_copy(z_hbm.at[idx_d.at[j]], rows_dx, semx)

    def wait(rows_sx, rows_dx, semx):
        pltpu.make_async_copy(z_hbm.at[pl.ds(0, CHUNK)], rows_sx, semx).wait()
        pltpu.make_async_copy(z_hbm.at[pl.ds(0, CHUNK)], rows_dx, semx).wait()

    def compute_chunk(j, rows_sx, rows_dx):
        for g in range(GROUPS):

            def edge_body(e16, vec):
                e = jnp.int32(g * 16) + e16
                acc = jnp.zeros((16,), jnp.float32)
                for k in range(D_FEAT // 16):
                    s = rows_sx[e, pl.ds(k * 16, 16)]
                    d = rows_dx[e, pl.ds(k * 16, 16)]
                    t = s - d + jnp.float32(1e-6)
                    acc = acc + t * t
                tot = jnp.sum(acc)
                return jnp.where(lane == e16, tot, vec)

            vec = lax.fori_loop(0, 16, edge_body,
                                jnp.zeros((16,), jnp.float32), unroll=4)
            outb[pl.ds(g * 16, 16)] = _tanh_neg_sqrt(vec)
        base = (wid * NCHUNK + j) * CHUNK
        pltpu.sync_copy(outb, out_hbm.at[pl.ds(base, CHUNK)])

    issue(0, rows_sa, rows_da, sem_a)
    issue(1, rows_sb, rows_db, sem_b)

    def pair_body(i, carry):
        j0 = jnp.int32(2) * i
        wait(rows_sa, rows_da, sem_a)
        compute_chunk(j0, rows_sa, rows_da)
        issue(j0 + 2, rows_sa, rows_da, sem_a)
        wait(rows_sb, rows_db, sem_b)
        compute_chunk(j0 + 1, rows_sb, rows_db)

        @pl.when(i < (NCHUNK - 1) // 2 - 1)
        def _():
            issue(j0 + 3, rows_sb, rows_db, sem_b)

        return carry

    lax.fori_loop(0, (NCHUNK - 1) // 2, pair_body, jnp.int32(0))
    wait(rows_sa, rows_da, sem_a)
    compute_chunk(jnp.int32(NCHUNK - 1), rows_sa, rows_da)


def kernel(z, edge_index):
    ei = edge_index.astype(jnp.int32)
    src = ei[0].reshape(NW, NCHUNK, CHUNK)
    dst = ei[1].reshape(NW, NCHUNK, CHUNK)
    return _sc_kernel(src, dst, z)


# speedup vs baseline: 7.8003x; 1.1418x over previous
import functools

import jax
import jax.numpy as jnp
from jax import lax
from jax.experimental import pallas as pl
from jax.experimental.pallas import tpu as pltpu
from jax.experimental.pallas import tpu_sc as plsc

N_NODES = 10000
D_FEAT = 128
N_EDGES = 320000

NC = 2
NS = 16
NW = NC * NS
EDGES_PER_WORKER = N_EDGES // NW
CHUNK = 80
NCHUNK = EDGES_PER_WORKER // CHUNK
GROUPS = CHUNK // 16


def _tanh_neg_sqrt(acc):
    x = jnp.maximum(acc, jnp.float32(1e-30))
    i = plsc.bitcast(x, jnp.int32)
    i = jnp.int32(0x5F3759DF) - (i >> 1)
    y = plsc.bitcast(i, jnp.float32)
    half_x = jnp.float32(0.5) * x
    for _ in range(3):
        y = y * (jnp.float32(1.5) - half_x * y * y)
    dist = x * y
    u = jnp.exp(jnp.float32(-2.0) * dist)
    return (u - jnp.float32(1.0)) / (u + jnp.float32(1.0))


@functools.partial(
    pl.kernel,
    mesh=plsc.VectorSubcoreMesh(core_axis_name="c", subcore_axis_name="s"),
    out_type=jax.ShapeDtypeStruct((N_EDGES,), jnp.float32),
    compiler_params=pltpu.CompilerParams(needs_layout_passes=False,
                                         use_tc_tiling_on_sc=False),
    scratch_types=[
        pltpu.VMEM((NCHUNK, CHUNK), jnp.int32),
        pltpu.VMEM((NCHUNK, CHUNK), jnp.int32),
        pltpu.VMEM((CHUNK, D_FEAT // 2), jnp.int32),
        pltpu.VMEM((CHUNK, D_FEAT // 2), jnp.int32),
        pltpu.VMEM((CHUNK, D_FEAT // 2), jnp.int32),
        pltpu.VMEM((CHUNK, D_FEAT // 2), jnp.int32),
        pltpu.VMEM((CHUNK,), jnp.float32),
        pltpu.SemaphoreType.DMA,
        pltpu.SemaphoreType.DMA,
    ],
)
def _sc_kernel(src_hbm, dst_hbm, z_hbm, out_hbm,
               idx_s, idx_d, rows_sa, rows_da, rows_sb, rows_db,
               outb, sem_a, sem_b):
    wid = lax.axis_index("s") * NC + lax.axis_index("c")
    pltpu.sync_copy(src_hbm.at[wid], idx_s)
    pltpu.sync_copy(dst_hbm.at[wid], idx_d)

    lane = lax.iota(jnp.int32, 16)

    def issue(j, rows_sx, rows_dx, semx):
        pltpu.async_copy(z_hbm.at[idx_s.at[j]], rows_sx, semx)
        pltpu.async_copy(z_hbm.at[idx_d.at[j]], rows_dx, semx)

    def wait(rows_sx, rows_dx, semx):
        pltpu.make_async_copy(z_hbm.at[pl.ds(0, CHUNK)], rows_sx, semx).wait()
        pltpu.make_async_copy(z_hbm.at[pl.ds(0, CHUNK)], rows_dx, semx).wait()

    def compute_chunk(j, rows_sx, rows_dx):
        for g in range(GROUPS):

            def edge_body(e16, vec):
                e = jnp.int32(g * 16) + e16
                acc = jnp.zeros((32,), jnp.bfloat16)
                for k in range(D_FEAT // 32):
                    s = plsc.bitcast(rows_sx[e, pl.ds(k * 16, 16)],
                                     jnp.bfloat16)
                    d = plsc.bitcast(rows_dx[e, pl.ds(k * 16, 16)],
                                     jnp.bfloat16)
                    t = s - d + jnp.bfloat16(1e-6)
                    acc = acc + t * t
                a0, a1 = plsc.unpack(acc, format=plsc.PackFormat.INTERLEAVED)
                tot = jnp.sum(a0 + a1)
                return jnp.where(lane == e16, tot, vec)

            vec = lax.fori_loop(0, 16, edge_body,
                                jnp.zeros((16,), jnp.float32), unroll=4)
            outb[pl.ds(g * 16, 16)] = _tanh_neg_sqrt(vec)
        base = (wid * NCHUNK + j) * CHUNK
        pltpu.sync_copy(outb, out_hbm.at[pl.ds(base, CHUNK)])

    issue(0, rows_sa, rows_da, sem_a)
    issue(1, rows_sb, rows_db, sem_b)

    def pair_body(i, carry):
        j0 = jnp.int32(2) * i
        wait(rows_sa, rows_da, sem_a)
        compute_chunk(j0, rows_sa, rows_da)
        issue(j0 + 2, rows_sa, rows_da, sem_a)
        wait(rows_sb, rows_db, sem_b)
        compute_chunk(j0 + 1, rows_sb, rows_db)

        @pl.when(i < (NCHUNK - 1) // 2 - 1)
        def _():
            issue(j0 + 3, rows_sb, rows_db, sem_b)

        return carry

    lax.fori_loop(0, (NCHUNK - 1) // 2, pair_body, jnp.int32(0))
    wait(rows_sa, rows_da, sem_a)
    compute_chunk(jnp.int32(NCHUNK - 1), rows_sa, rows_da)


def kernel(z, edge_index):
    ei = edge_index.astype(jnp.int32)
    src = ei[0].reshape(NW, NCHUNK, CHUNK)
    dst = ei[1].reshape(NW, NCHUNK, CHUNK)
    zw = lax.bitcast_convert_type(
        z.astype(jnp.bfloat16).reshape(N_NODES, D_FEAT // 2, 2), jnp.int32)
    return _sc_kernel(src, dst, zw)
